# baseline (speedup 1.0000x reference)
import functools

import jax
import jax.numpy as jnp
from jax import lax
from jax.experimental import pallas as pl
from jax.experimental.pallas import tpu as pltpu
from jax.experimental.pallas import tpu_sc as plsc

_B, _L, _V, _D, _H, _O = 16, 512, 1000000, 128, 128, 128

_NW = 32
_CHUNK = 128
_BH = _B // 2
_TILES_PER_ROW = _L // _CHUNK
_NDOC = 2
_GH = _BH // _NDOC


@functools.cache
def _sc_gather_half(h):
    def body(table_hbm, wid_hbm, xw_hbm, idx_v, rows_v, sem):
        wid = lax.axis_index("s") * 2 + lax.axis_index("c")
        row = h * _BH + wid // _TILES_PER_ROW
        col = (wid % _TILES_PER_ROW) * _CHUNK
        pltpu.sync_copy(wid_hbm.at[row, pl.ds(col, _CHUNK)], idx_v)
        pltpu.async_copy(table_hbm.at[idx_v], rows_v, sem).wait()
        pltpu.sync_copy(rows_v, xw_hbm.at[wid])

    return pl.kernel(
        body,
        out_type=jax.ShapeDtypeStruct((_NW, _CHUNK, _D), jnp.float32),
        mesh=plsc.VectorSubcoreMesh(core_axis_name="c", subcore_axis_name="s"),
        scratch_types=[
            pltpu.VMEM((_CHUNK,), jnp.int32),
            pltpu.VMEM((_CHUNK, _D), jnp.float32),
            pltpu.SemaphoreType.DMA,
        ],
    )


def _tc_gcn_body(h, *refs, has_prev):
    if has_prev:
        (im_ref, a_ref, xw_ref, me_ref, w1_ref, b1_ref, w2_ref,
         b2_ref, _prev_ref, o_ref) = refs
    else:
        (im_ref, a_ref, xw_ref, me_ref, w1_ref, b1_ref, w2_ref,
         b2_ref, o_ref) = refs
    g = pl.program_id(0)
    me = me_ref[...]
    diff = me[1:2] - me[0:1]
    w1 = w1_ref[...]
    b1v = b1_ref[...]
    w2 = w2_ref[...]
    b2v = b2_ref[...]
    for j in range(_NDOC):
        doc = h * _BH + _NDOC * g + j
        m = im_ref[pl.ds(doc, 1), :].astype(jnp.float32)
        contrib = lax.dot_general(
            m, diff, (((0,), (0,)), ((), ())),
            preferred_element_type=jnp.float32)
        x = xw_ref[j] + me[0:1] + contrib
        a_mat = a_ref[j]
        ax = jnp.dot(a_mat, x, preferred_element_type=jnp.float32)
        h1 = jnp.maximum(
            jnp.dot(ax, w1, preferred_element_type=jnp.float32) + b1v, 0.0)
        ah = jnp.dot(a_mat, h1, preferred_element_type=jnp.float32)
        o_ref[j] = jnp.dot(ah, w2, preferred_element_type=jnp.float32) + b2v


def _tc_gcn_half(h, i_mask, paris_mat, xw_h, mask_embedding, W1, b1, W2, b2,
                 prev=None):
    blk_off = h * _GH
    in_specs = [
        pl.BlockSpec((_B, _L), lambda g: (0, 0)),
        pl.BlockSpec((_NDOC, _L, _L), lambda g: (g + blk_off, 0, 0)),
        pl.BlockSpec((_NDOC, _L, _D), lambda g: (g, 0, 0)),
        pl.BlockSpec((2, _D), lambda g: (0, 0)),
        pl.BlockSpec((_D, _H), lambda g: (0, 0)),
        pl.BlockSpec((1, _H), lambda g: (0, 0)),
        pl.BlockSpec((_H, _O), lambda g: (0, 0)),
        pl.BlockSpec((1, _O), lambda g: (0, 0)),
    ]
    args = [i_mask, paris_mat, xw_h, mask_embedding, W1, b1, W2, b2]
    aliases = {}
    if prev is not None:
        in_specs.append(pl.BlockSpec(memory_space=pl.ANY))
        args.append(prev)
        aliases = {8: 0}
    body = functools.partial(_tc_gcn_body, h, has_prev=prev is not None)
    return pl.pallas_call(
        body,
        grid=(_GH,),
        in_specs=in_specs,
        out_specs=pl.BlockSpec((_NDOC, _L, _O), lambda g: (g + blk_off, 0, 0)),
        out_shape=jax.ShapeDtypeStruct((_B, _L, _O), jnp.float32),
        input_output_aliases=aliases,
        compiler_params=pltpu.CompilerParams(
            dimension_semantics=("arbitrary",)),
    )(*args)


def kernel(words2ids, i_mask, paris_mat, w_embedding, mask_embedding, W1, b1, W2, b2):
    b1r = b1.reshape(1, _H)
    b2r = b2.reshape(1, _O)
    xw0 = _sc_gather_half(0)(w_embedding, words2ids).reshape(_BH, _L, _D)
    xw1 = _sc_gather_half(1)(w_embedding, words2ids).reshape(_BH, _L, _D)
    out0 = _tc_gcn_half(0, i_mask, paris_mat, xw0, mask_embedding,
                        W1, b1r, W2, b2r)
    return _tc_gcn_half(1, i_mask, paris_mat, xw1, mask_embedding,
                        W1, b1r, W2, b2r, prev=out0)


# speedup vs baseline: 1.0414x; 1.0414x over previous
import functools

import jax
import jax.numpy as jnp
from jax import lax
from jax.experimental import pallas as pl
from jax.experimental.pallas import tpu as pltpu
from jax.experimental.pallas import tpu_sc as plsc

_B, _L, _V, _D, _H, _O = 16, 512, 1000000, 128, 128, 128

_NW = 32
_CHUNK = 128
_TOK = _B * _L
_PER_W = _TOK // _NW
_NCH = _PER_W // _CHUNK
_ROWS_PER_B = _L // _PER_W


def _sc_gather_body(table_hbm, wid_hbm, xw_hbm, idxw_v, rows_w, sem):
    wid = lax.axis_index("s") * 2 + lax.axis_index("c")
    row = wid // _ROWS_PER_B
    col = (wid % _ROWS_PER_B) * _PER_W
    for k in range(_NCH):
        pltpu.sync_copy(wid_hbm.at[row, pl.ds(col + k * _CHUNK, _CHUNK)],
                        idxw_v.at[k])
    cps = [pltpu.async_copy(table_hbm.at[idxw_v.at[k]], rows_w.at[k], sem)
           for k in range(_NCH)]
    for c in cps:
        c.wait()
    pltpu.sync_copy(rows_w, xw_hbm.at[wid])


@functools.cache
def _sc_gather():
    return pl.kernel(
        _sc_gather_body,
        out_type=jax.ShapeDtypeStruct((_NW, _NCH, _CHUNK, _D), jnp.float32),
        mesh=plsc.VectorSubcoreMesh(core_axis_name="c", subcore_axis_name="s"),
        scratch_types=[
            pltpu.VMEM((_NCH, _CHUNK), jnp.int32),
            pltpu.VMEM((_NCH, _CHUNK, _D), jnp.float32),
            pltpu.SemaphoreType.DMA,
        ],
    )


_NDOC = 2


def _tc_gcn_body(im_ref, a_ref, xw_ref, me_ref, w1_ref, b1_ref, w2_ref,
                 b2_ref, o_ref):
    g = pl.program_id(0)
    me = me_ref[...]
    diff = me[1:2] - me[0:1]
    w1 = w1_ref[...]
    b1v = b1_ref[...]
    w2 = w2_ref[...]
    b2v = b2_ref[...]
    for j in range(_NDOC):
        m = im_ref[pl.ds(_NDOC * g + j, 1), :].astype(jnp.float32)
        contrib = lax.dot_general(
            m, diff, (((0,), (0,)), ((), ())),
            preferred_element_type=jnp.float32)
        x = xw_ref[j] + me[0:1] + contrib
        a_mat = a_ref[j]
        ax = jnp.dot(a_mat, x, preferred_element_type=jnp.float32)
        h = jnp.maximum(
            jnp.dot(ax, w1, preferred_element_type=jnp.float32) + b1v, 0.0)
        ah = jnp.dot(a_mat, h, preferred_element_type=jnp.float32)
        o_ref[j] = jnp.dot(ah, w2, preferred_element_type=jnp.float32) + b2v


def _tc_gcn(i_mask, paris_mat, xw, mask_embedding, W1, b1, W2, b2):
    return pl.pallas_call(
        _tc_gcn_body,
        grid=(_B // _NDOC,),
        in_specs=[
            pl.BlockSpec((_B, _L), lambda g: (0, 0)),
            pl.BlockSpec((_NDOC, _L, _L), lambda g: (g, 0, 0)),
            pl.BlockSpec((_NDOC, _L, _D), lambda g: (g, 0, 0)),
            pl.BlockSpec((2, _D), lambda g: (0, 0)),
            pl.BlockSpec((_D, _H), lambda g: (0, 0)),
            pl.BlockSpec((1, _H), lambda g: (0, 0)),
            pl.BlockSpec((_H, _O), lambda g: (0, 0)),
            pl.BlockSpec((1, _O), lambda g: (0, 0)),
        ],
        out_specs=pl.BlockSpec((_NDOC, _L, _O), lambda g: (g, 0, 0)),
        out_shape=jax.ShapeDtypeStruct((_B, _L, _O), jnp.float32),
        compiler_params=pltpu.CompilerParams(
            dimension_semantics=("arbitrary",)),
    )(i_mask, paris_mat, xw, mask_embedding, W1, b1, W2, b2)


def kernel(words2ids, i_mask, paris_mat, w_embedding, mask_embedding, W1, b1, W2, b2):
    xw = _sc_gather()(w_embedding, words2ids).reshape(_B, _L, _D)
    return _tc_gcn(i_mask, paris_mat, xw, mask_embedding,
                   W1, b1.reshape(1, _H), W2, b2.reshape(1, _O))
